# baseline (speedup 1.0000x reference)
import functools

import jax
import jax.numpy as jnp
from jax import lax
from jax.experimental import pallas as pl
from jax.experimental.pallas import tpu as pltpu
from jax.experimental.pallas import tpu_sc as plsc


@functools.lru_cache(maxsize=None)
def _make_copy_kernel(seq_len: int, dim: int):
    info = plsc.get_sparse_core_info()
    nc, ns = info.num_cores, info.num_subcores
    nw = nc * ns
    assert seq_len % nw == 0
    rows_per_w = seq_len // nw

    chunk = 32
    nbuf = 3
    assert rows_per_w % chunk == 0
    nchunks = rows_per_w // chunk

    mesh = plsc.VectorSubcoreMesh(core_axis_name="c", subcore_axis_name="s")

    @functools.partial(
        pl.kernel,
        out_type=jax.ShapeDtypeStruct((seq_len, dim), jnp.float32),
        mesh=mesh,
        scratch_types=[
            pltpu.VMEM((nbuf, chunk, dim), jnp.float32),
            [pltpu.SemaphoreType.DMA] * nbuf,
            [pltpu.SemaphoreType.DMA] * nbuf,
        ],
    )
    def copy_kernel(table_hbm, out_hbm, bufs, in_sems, out_sems):
        wid = lax.axis_index("s") * nc + lax.axis_index("c")
        base = wid * rows_per_w

        def in_cp(i):
            b = i % nbuf
            return pltpu.make_async_copy(
                table_hbm.at[pl.ds(base + i * chunk, chunk)], bufs.at[b], in_sems[b]
            )

        def out_cp(i):
            b = i % nbuf
            return pltpu.make_async_copy(
                bufs.at[b], out_hbm.at[pl.ds(base + i * chunk, chunk)], out_sems[b]
            )

        in_cp(0).start()
        for i in range(nchunks):
            nxt = i + 1
            if nxt < nchunks:
                if nxt >= nbuf:
                    out_cp(nxt - nbuf).wait()
                in_cp(nxt).start()
            in_cp(i).wait()
            out_cp(i).start()
        for i in range(max(0, nchunks - nbuf), nchunks):
            out_cp(i).wait()

    return copy_kernel


def kernel(inputs, table):
    seq_len = inputs.shape[1]
    return _make_copy_kernel(seq_len, table.shape[1])(table)


# speedup vs baseline: 1.0435x; 1.0435x over previous
import functools

import jax
import jax.numpy as jnp
from jax import lax
from jax.experimental import pallas as pl
from jax.experimental.pallas import tpu as pltpu
from jax.experimental.pallas import tpu_sc as plsc


@functools.lru_cache(maxsize=None)
def _make_copy_kernel(seq_len: int, dim: int):
    info = plsc.get_sparse_core_info()
    nc, ns = info.num_cores, info.num_subcores
    nw = nc * ns
    assert seq_len % nw == 0
    rows_per_w = seq_len // nw

    chunk = 32
    nbuf = 3
    lookahead = 1
    assert rows_per_w % chunk == 0
    nchunks = rows_per_w // chunk

    mesh = plsc.VectorSubcoreMesh(core_axis_name="c", subcore_axis_name="s")

    @functools.partial(
        pl.kernel,
        out_type=jax.ShapeDtypeStruct((seq_len, dim), jnp.float32),
        mesh=mesh,
        scratch_types=[
            pltpu.VMEM((nbuf, chunk, dim), jnp.float32),
            pltpu.SemaphoreType.DMA,
            pltpu.SemaphoreType.DMA,
        ],
    )
    def copy_kernel(table_hbm, out_hbm, bufs, in_sem, out_sem):
        wid = lax.axis_index("s") * nc + lax.axis_index("c")
        base = wid * rows_per_w

        def in_cp(i, b):
            return pltpu.make_async_copy(
                table_hbm.at[pl.ds(base + i * chunk, chunk)], bufs.at[b], in_sem
            )

        def out_cp(i, b):
            return pltpu.make_async_copy(
                bufs.at[b], out_hbm.at[pl.ds(base + i * chunk, chunk)], out_sem
            )

        started = 0
        for i in range(nchunks):
            while started < min(i + 1 + lookahead, nchunks):
                j = started
                if j >= nbuf:
                    out_cp(j - nbuf, j % nbuf).wait()
                in_cp(j, j % nbuf).start()
                started += 1
            in_cp(i, i % nbuf).wait()
            out_cp(i, i % nbuf).start()
        for i in range(max(0, nchunks - nbuf), nchunks):
            out_cp(i, i % nbuf).wait()

    return copy_kernel


def kernel(inputs, table):
    seq_len = inputs.shape[1]
    return _make_copy_kernel(seq_len, table.shape[1])(table)
